# Initial kernel scaffold; baseline (speedup 1.0000x reference)
import jax
import jax.numpy as jnp
from jax.experimental import pallas as pl


def kernel(sim_matrix, pids):
    raise NotImplementedError("write your pallas kernel here")


# speedup vs baseline: 10.6779x; 10.6779x over previous
import functools

import jax
import jax.numpy as jnp
import numpy as np
from jax.experimental import pallas as pl
from jax.experimental.pallas import tpu as pltpu

_TEMPERATURE = 0.02
_K = 10
_NEG = float(np.finfo(np.float32).min)


def _fused_body(row_ref, col_ref, unc_ref, ent_ref, *, k: int, max_ent: float):
    X = row_ref[...]
    Y = col_ref[...]
    blk = X.shape[0]

    s = X * (1.0 / _TEMPERATURE)
    m = jnp.max(s, axis=1, keepdims=True)
    sm = s - m
    e = jnp.exp(sm)
    Z = jnp.sum(e, axis=1, keepdims=True)
    S1 = jnp.sum(sm * e, axis=1, keepdims=True)
    ent = (jnp.log(Z) - S1 / Z)[:, 0] * (1.0 / max_ent)

    xm = X
    tr = None
    for _ in range(k):
        tr = jnp.max(xm, axis=1, keepdims=True)
        xm = jnp.where(xm >= tr, _NEG, xm)

    ym = Y
    tc = None
    for _ in range(k):
        tc = jnp.max(ym, axis=0, keepdims=True)
        ym = jnp.where(ym >= tc, _NEG, ym)

    R = (X >= tr).astype(jnp.float32)
    C = (Y >= tc).astype(jnp.float32)
    P = jax.lax.dot(R, C, preferred_element_type=jnp.float32)
    ii = jax.lax.broadcasted_iota(jnp.int32, (blk, blk), 0)
    jj = jax.lax.broadcasted_iota(jnp.int32, (blk, blk), 1)
    matches = jnp.sum(jnp.where(ii == jj, P, 0.0), axis=1)

    ra = matches * (1.0 / k)
    unc_ref[...] = (1.0 - ra) * 0.5 + ent * 0.5
    ent_ref[...] = ent


def kernel(sim_matrix, pids):
    del pids
    B = sim_matrix.shape[0]
    blk = 128
    k = min(_K, B)
    max_ent = float(np.log(B + 1e-10))
    grid = B // blk
    unc, ent = pl.pallas_call(
        functools.partial(_fused_body, k=k, max_ent=max_ent),
        grid=(grid,),
        in_specs=[
            pl.BlockSpec((blk, B), lambda i: (i, 0)),
            pl.BlockSpec((B, blk), lambda i: (0, i)),
        ],
        out_specs=[
            pl.BlockSpec((blk,), lambda i: (i,)),
            pl.BlockSpec((blk,), lambda i: (i,)),
        ],
        out_shape=[
            jax.ShapeDtypeStruct((B,), jnp.float32),
            jax.ShapeDtypeStruct((B,), jnp.float32),
        ],
    )(sim_matrix, sim_matrix)
    return (unc, ent)


# speedup vs baseline: 14.3113x; 1.3403x over previous
import functools

import jax
import jax.numpy as jnp
import numpy as np
from jax.experimental import pallas as pl
from jax.experimental.pallas import tpu as pltpu

_TEMPERATURE = 0.02
_K = 10
_NEG = float(np.finfo(np.float32).min)


def _fused_body(row_ref, col_ref, unc_ref, ent_ref, *, k: int, max_ent: float):
    X = row_ref[...]
    Y = col_ref[...]
    blk = X.shape[0]

    s = X * (1.0 / _TEMPERATURE)
    m = jnp.max(s, axis=1, keepdims=True)
    sm = s - m
    e = jnp.exp(sm)
    Z = jnp.sum(e, axis=1, keepdims=True)
    S1 = jnp.sum(sm * e, axis=1, keepdims=True)
    ent = (jnp.log(Z) - S1 / Z)[:, 0] * (1.0 / max_ent)

    xm = X
    tr = None
    for _ in range(k):
        tr = jnp.max(xm, axis=1, keepdims=True)
        xm = jnp.where(xm >= tr, _NEG, xm)

    ym = Y
    tc = None
    for _ in range(k):
        tc = jnp.max(ym, axis=0, keepdims=True)
        ym = jnp.where(ym >= tc, _NEG, ym)

    R = (X >= tr).astype(jnp.float32)
    C = (Y >= tc).astype(jnp.float32)
    P = jax.lax.dot(R, C, preferred_element_type=jnp.float32)
    ii = jax.lax.broadcasted_iota(jnp.int32, (blk, blk), 0)
    jj = jax.lax.broadcasted_iota(jnp.int32, (blk, blk), 1)
    matches = jnp.sum(jnp.where(ii == jj, P, 0.0), axis=1)

    ra = matches * (1.0 / k)
    unc_ref[...] = (1.0 - ra) * 0.5 + ent * 0.5
    ent_ref[...] = ent


def kernel(sim_matrix, pids):
    del pids
    B = sim_matrix.shape[0]
    blk = 256
    k = min(_K, B)
    max_ent = float(np.log(B + 1e-10))
    grid = B // blk
    unc, ent = pl.pallas_call(
        functools.partial(_fused_body, k=k, max_ent=max_ent),
        grid=(grid,),
        in_specs=[
            pl.BlockSpec((blk, B), lambda i: (i, 0)),
            pl.BlockSpec((B, blk), lambda i: (0, i)),
        ],
        out_specs=[
            pl.BlockSpec((blk,), lambda i: (i,)),
            pl.BlockSpec((blk,), lambda i: (i,)),
        ],
        out_shape=[
            jax.ShapeDtypeStruct((B,), jnp.float32),
            jax.ShapeDtypeStruct((B,), jnp.float32),
        ],
    )(sim_matrix, sim_matrix)
    return (unc, ent)


# speedup vs baseline: 14.6519x; 1.0238x over previous
import functools

import jax
import jax.numpy as jnp
import numpy as np
from jax.experimental import pallas as pl
from jax.experimental.pallas import tpu as pltpu

_TEMPERATURE = 0.02
_K = 10
_NEG = float(np.finfo(np.float32).min)


def _fused_body(row_ref, col_ref, unc_ref, ent_ref, *, k: int, max_ent: float):
    X = row_ref[...]
    Y = col_ref[...]
    blk = X.shape[0]

    s = X * (1.0 / _TEMPERATURE)
    m = jnp.max(s, axis=1, keepdims=True)
    sm = s - m
    e = jnp.exp(sm)
    Z = jnp.sum(e, axis=1, keepdims=True)
    S1 = jnp.sum(sm * e, axis=1, keepdims=True)
    ent = (jnp.log(Z) - S1 / Z)[:, 0] * (1.0 / max_ent)

    xm = X
    tr = None
    for _ in range(k):
        tr = jnp.max(xm, axis=1, keepdims=True)
        xm = jnp.where(xm >= tr, _NEG, xm)

    ym = Y
    tc = None
    for _ in range(k):
        tc = jnp.max(ym, axis=0, keepdims=True)
        ym = jnp.where(ym >= tc, _NEG, ym)

    R = (X >= tr).astype(jnp.float32)
    C = (Y >= tc).astype(jnp.float32)
    P = jax.lax.dot(R, C, preferred_element_type=jnp.float32)
    ii = jax.lax.broadcasted_iota(jnp.int32, (blk, blk), 0)
    jj = jax.lax.broadcasted_iota(jnp.int32, (blk, blk), 1)
    matches = jnp.sum(jnp.where(ii == jj, P, 0.0), axis=1)

    ra = matches * (1.0 / k)
    unc_ref[...] = (1.0 - ra) * 0.5 + ent * 0.5
    ent_ref[...] = ent


def kernel(sim_matrix, pids):
    del pids
    B = sim_matrix.shape[0]
    blk = 512
    k = min(_K, B)
    max_ent = float(np.log(B + 1e-10))
    grid = B // blk
    unc, ent = pl.pallas_call(
        functools.partial(_fused_body, k=k, max_ent=max_ent),
        grid=(grid,),
        in_specs=[
            pl.BlockSpec((blk, B), lambda i: (i, 0)),
            pl.BlockSpec((B, blk), lambda i: (0, i)),
        ],
        out_specs=[
            pl.BlockSpec((blk,), lambda i: (i,)),
            pl.BlockSpec((blk,), lambda i: (i,)),
        ],
        out_shape=[
            jax.ShapeDtypeStruct((B,), jnp.float32),
            jax.ShapeDtypeStruct((B,), jnp.float32),
        ],
    )(sim_matrix, sim_matrix)
    return (unc, ent)
